# baseline (speedup 1.0000x reference)
import jax
import jax.numpy as jnp
from jax.experimental import pallas as pl
from jax.experimental.pallas import tpu as pltpu

_B, _S, _E = 4, 2048, 4096
_P = 10
_SBLK = 1024
_EPW = 5 * 32 * 128


def _body(x_ref, w_ref, b_ref, p_ref, logits_ref, ep_ref, acc_ref):
    si = pl.program_id(1)
    blk = jnp.max(x_ref[...], axis=1)

    @pl.when(si == 0)
    def _init():
        acc_ref[...] = blk

    @pl.when(si > 0)
    def _acc():
        acc_ref[...] = jnp.maximum(acc_ref[...], blk)

    @pl.when(si == pl.num_programs(1) - 1)
    def _final():
        logits = jax.lax.dot_general(
            acc_ref[...], w_ref[...],
            (((1,), (1,)), ((), ())),
            preferred_element_type=jnp.float32,
        ) + b_ref[...]
        logits_ref[0] = logits
        iota = jax.lax.broadcasted_iota(jnp.int32, (1, _P), 1)
        m = jnp.max(logits, axis=1, keepdims=True)
        idx = jnp.min(jnp.where(logits == m, iota, _P), axis=1, keepdims=True)
        onehot = (iota == idx).astype(jnp.float32)
        for t in range(2):
            ep_ref[:, t, :] = jax.lax.dot_general(
                onehot, p_ref[t],
                (((1,), (0,)), ((), ())),
                preferred_element_type=jnp.float32,
            )


def kernel(x_embed, prompt, W, b):
    p2 = prompt.reshape(2, _P, _EPW)
    b2 = b.reshape(1, _P)
    logits, ep = pl.pallas_call(
        _body,
        grid=(_B, _S // _SBLK),
        in_specs=[
            pl.BlockSpec((1, _SBLK, _E), lambda bi, si: (bi, si, 0)),
            pl.BlockSpec((_P, _E), lambda bi, si: (0, 0)),
            pl.BlockSpec((1, _P), lambda bi, si: (0, 0)),
            pl.BlockSpec((2, _P, _EPW), lambda bi, si: (0, 0, 0)),
        ],
        out_specs=[
            pl.BlockSpec((1, 1, _P), lambda bi, si: (bi, 0, 0)),
            pl.BlockSpec((1, 2, _EPW), lambda bi, si: (bi, 0, 0)),
        ],
        out_shape=[
            jax.ShapeDtypeStruct((_B, 1, _P), jnp.float32),
            jax.ShapeDtypeStruct((_B, 2, _EPW), jnp.float32),
        ],
        scratch_shapes=[pltpu.VMEM((1, _E), jnp.float32)],
        compiler_params=pltpu.CompilerParams(
            dimension_semantics=("parallel", "arbitrary"),
        ),
    )(x_embed, W, b2, p2)
    e_prompt = ep.reshape(1, _B, 2, 5, 32, 128)
    return (logits.reshape(_B, _P), e_prompt)


# speedup vs baseline: 1.0273x; 1.0273x over previous
import jax
import jax.numpy as jnp
from jax.experimental import pallas as pl
from jax.experimental.pallas import tpu as pltpu

_B, _S, _E = 4, 2048, 4096
_P = 10
_R = _S // 8
_CH = 64
_EPW = 5 * 32 * 128


def _body(x_ref, w_ref, b_ref, p_ref, logits_ref, ep_ref, acc_ref):
    si = pl.program_id(1)
    part = jnp.max(x_ref[...], axis=1)

    @pl.when(si == 0)
    def _init():
        acc_ref[...] = part

    @pl.when(si > 0)
    def _acc():
        acc_ref[...] = jnp.maximum(acc_ref[...], part)

    @pl.when(si == pl.num_programs(1) - 1)
    def _final():
        red = jnp.max(acc_ref[0], axis=0, keepdims=True)
        logits = jax.lax.dot_general(
            red, w_ref[...],
            (((1,), (1,)), ((), ())),
            preferred_element_type=jnp.float32,
        ) + b_ref[...]
        logits_ref[0] = logits
        iota = jax.lax.broadcasted_iota(jnp.int32, (1, _P), 1)
        m = jnp.max(logits, axis=1, keepdims=True)
        idx = jnp.min(jnp.where(logits == m, iota, _P), axis=1, keepdims=True)
        onehot = (iota == idx).astype(jnp.float32)
        for t in range(2):
            ep_ref[:, t, :] = jax.lax.dot_general(
                onehot, p_ref[t],
                (((1,), (0,)), ((), ())),
                preferred_element_type=jnp.float32,
            )


def kernel(x_embed, prompt, W, b):
    x4 = x_embed.reshape(_B, _R, 8, _E)
    p2 = prompt.reshape(2, _P, _EPW)
    b2 = b.reshape(1, _P)
    logits, ep = pl.pallas_call(
        _body,
        grid=(_B, _R // _CH),
        in_specs=[
            pl.BlockSpec((1, _CH, 8, _E), lambda bi, si: (bi, si, 0, 0)),
            pl.BlockSpec((_P, _E), lambda bi, si: (0, 0)),
            pl.BlockSpec((1, _P), lambda bi, si: (0, 0)),
            pl.BlockSpec((2, _P, _EPW), lambda bi, si: (0, 0, 0)),
        ],
        out_specs=[
            pl.BlockSpec((1, 1, _P), lambda bi, si: (bi, 0, 0)),
            pl.BlockSpec((1, 2, _EPW), lambda bi, si: (bi, 0, 0)),
        ],
        out_shape=[
            jax.ShapeDtypeStruct((_B, 1, _P), jnp.float32),
            jax.ShapeDtypeStruct((_B, 2, _EPW), jnp.float32),
        ],
        scratch_shapes=[pltpu.VMEM((1, 8, _E), jnp.float32)],
        compiler_params=pltpu.CompilerParams(
            dimension_semantics=("parallel", "arbitrary"),
        ),
    )(x4, W, b2, p2)
    e_prompt = ep.reshape(1, _B, 2, 5, 32, 128)
    return (logits.reshape(_B, _P), e_prompt)


# speedup vs baseline: 1.0560x; 1.0279x over previous
import jax
import jax.numpy as jnp
from jax.experimental import pallas as pl
from jax.experimental.pallas import tpu as pltpu

_B, _S, _E = 4, 2048, 4096
_P = 10
_R = _S // 8
_CH = 32
_EPW = 5 * 32 * 128


def _body(xa_ref, xb_ref, w_ref, b_ref, p_ref, logits_ref, ep_ref, acc_ref):
    si = pl.program_id(1)
    part = jnp.maximum(
        jnp.max(xa_ref[0, 0], axis=0),
        jnp.max(xb_ref[0, 0], axis=0),
    )[None]

    @pl.when(si == 0)
    def _init():
        acc_ref[...] = part

    @pl.when(si > 0)
    def _acc():
        acc_ref[...] = jnp.maximum(acc_ref[...], part)

    @pl.when(si == pl.num_programs(1) - 1)
    def _final():
        red = jnp.max(acc_ref[0], axis=0, keepdims=True)
        logits = jax.lax.dot_general(
            red, w_ref[...],
            (((1,), (1,)), ((), ())),
            preferred_element_type=jnp.float32,
        ) + b_ref[...]
        logits_ref[0] = logits
        iota = jax.lax.broadcasted_iota(jnp.int32, (1, _P), 1)
        m = jnp.max(logits, axis=1, keepdims=True)
        idx = jnp.min(jnp.where(logits == m, iota, _P), axis=1, keepdims=True)
        onehot = (iota == idx).astype(jnp.float32)
        for t in range(2):
            ep_ref[:, t, :] = jax.lax.dot_general(
                onehot, p_ref[t],
                (((1,), (0,)), ((), ())),
                preferred_element_type=jnp.float32,
            )


def kernel(x_embed, prompt, W, b):
    x5 = x_embed.reshape(_B, 2, _R // 2, 8, _E)
    p2 = prompt.reshape(2, _P, _EPW)
    b2 = b.reshape(1, _P)
    logits, ep = pl.pallas_call(
        _body,
        grid=(_B, _R // 2 // _CH),
        in_specs=[
            pl.BlockSpec((1, 1, _CH, 8, _E), lambda bi, si: (bi, 0, si, 0, 0)),
            pl.BlockSpec((1, 1, _CH, 8, _E), lambda bi, si: (bi, 1, si, 0, 0)),
            pl.BlockSpec((_P, _E), lambda bi, si: (0, 0)),
            pl.BlockSpec((1, _P), lambda bi, si: (0, 0)),
            pl.BlockSpec((2, _P, _EPW), lambda bi, si: (0, 0, 0)),
        ],
        out_specs=[
            pl.BlockSpec((1, 1, _P), lambda bi, si: (bi, 0, 0)),
            pl.BlockSpec((1, 2, _EPW), lambda bi, si: (bi, 0, 0)),
        ],
        out_shape=[
            jax.ShapeDtypeStruct((_B, 1, _P), jnp.float32),
            jax.ShapeDtypeStruct((_B, 2, _EPW), jnp.float32),
        ],
        scratch_shapes=[pltpu.VMEM((1, 8, _E), jnp.float32)],
        compiler_params=pltpu.CompilerParams(
            dimension_semantics=("parallel", "arbitrary"),
        ),
    )(x5, x5, W, b2, p2)
    e_prompt = ep.reshape(1, _B, 2, 5, 32, 128)
    return (logits.reshape(_B, _P), e_prompt)
